# baseline (speedup 1.0000x reference)
import functools

import jax
import jax.numpy as jnp
from jax import lax
from jax.experimental import pallas as pl
from jax.experimental.pallas import tpu as pltpu
from jax.experimental.pallas import tpu_sc as plsc

_D = 128
_B = 1024
_FEW = 5
_N_ENT = _B * _FEW * 2
_N_TEM = 366

_info = plsc.get_sparse_core_info()
_NC, _NS = _info.num_cores, _info.num_subcores
_NW = _NC * _NS
_ENT_W = _N_ENT // _NW
_ENT_B = _ENT_W // (_FEW * 2)

_mesh = plsc.VectorSubcoreMesh(core_axis_name="c", subcore_axis_name="s")


_CHUNK = 80
_ENT_CH = _ENT_W // _CHUNK


@functools.partial(
    pl.kernel,
    mesh=_mesh,
    out_type=jax.ShapeDtypeStruct((_B, _FEW, 2, _D), jnp.float32),
    scratch_types=[
        pltpu.VMEM((_ENT_W,), jnp.int32),
        pltpu.VMEM((_ENT_W, _D), jnp.float32),
        pltpu.SemaphoreType.DMA,
    ],
)
def _ent_gather(ent_hbm, idx_hbm, out_a, idx_v, rows_v, sem):
    wid = lax.axis_index("s") * _NC + lax.axis_index("c")
    pltpu.sync_copy(idx_hbm.at[pl.ds(wid * _ENT_W, _ENT_W)], idx_v)
    pltpu.async_copy(ent_hbm.at[idx_v], rows_v, sem).wait()
    pltpu.sync_copy(rows_v.reshape(_ENT_B, _FEW, 2, _D),
                    out_a.at[pl.ds(wid * _ENT_B, _ENT_B)])


_TC_BLK = 1024


def _tem_body(idx_ref, tem_ref, out_ref):
    table = tem_ref[...]
    t_hi = table.astype(jnp.bfloat16)
    t_lo = (table - t_hi.astype(jnp.float32)).astype(jnp.bfloat16)
    for f in range(_FEW):
        col = idx_ref[:, f]
        onehot = (lax.broadcasted_iota(jnp.int32, (_TC_BLK, _N_TEM), 1)
                  == col[:, None]).astype(jnp.bfloat16)
        out_ref[:, f, :] = (
            jnp.dot(onehot, t_hi, preferred_element_type=jnp.float32)
            + jnp.dot(onehot, t_lo, preferred_element_type=jnp.float32))


_tem_lookup = pl.pallas_call(
    _tem_body,
    grid=(_B // _TC_BLK,),
    in_specs=[
        pl.BlockSpec((_TC_BLK, _FEW), lambda i: (i, 0)),
        pl.BlockSpec((_N_TEM, _D), lambda i: (0, 0)),
    ],
    out_specs=pl.BlockSpec((_TC_BLK, _FEW, _D), lambda i: (i, 0, 0)),
    out_shape=jax.ShapeDtypeStruct((_B, _FEW, _D), jnp.float32),
)


def kernel(idx, idx_tem, ent_emb, tem_emb):
    a = _ent_gather(ent_emb, idx.reshape(-1).astype(jnp.int32))
    b = _tem_lookup(idx_tem.astype(jnp.int32), tem_emb)
    return (a, b)


# speedup vs baseline: 1.0222x; 1.0222x over previous
import functools

import jax
import jax.numpy as jnp
from jax import lax
from jax.experimental import pallas as pl
from jax.experimental.pallas import tpu as pltpu
from jax.experimental.pallas import tpu_sc as plsc

_D = 128
_B = 1024
_FEW = 5
_N_ENT = _B * _FEW * 2
_N_TEM = 366

_info = plsc.get_sparse_core_info()
_NC, _NS = _info.num_cores, _info.num_subcores
_NW = _NC * _NS
_ENT_W = _N_ENT // _NW
_ENT_B = _ENT_W // (_FEW * 2)

_mesh = plsc.VectorSubcoreMesh(core_axis_name="c", subcore_axis_name="s")


@functools.partial(
    pl.kernel,
    mesh=_mesh,
    out_type=jax.ShapeDtypeStruct((_B, _FEW, 2, _D), jnp.float32),
    scratch_types=[
        pltpu.VMEM((_ENT_W,), jnp.int32),
        pltpu.VMEM((_ENT_W, _D), jnp.float32),
        pltpu.SemaphoreType.DMA,
    ],
)
def _ent_gather(ent_hbm, idx_hbm, out_a, idx_v, rows_v, sem):
    wid = lax.axis_index("s") * _NC + lax.axis_index("c")
    pltpu.sync_copy(idx_hbm.at[pl.ds(wid * _ENT_W, _ENT_W)], idx_v)
    pltpu.async_copy(ent_hbm.at[idx_v], rows_v, sem).wait()
    pltpu.sync_copy(rows_v.reshape(_ENT_B, _FEW, 2, _D),
                    out_a.at[pl.ds(wid * _ENT_B, _ENT_B)])


_TC_BLK = 512


def _tem_body(idx_ref, tem_ref, out_ref):
    table = tem_ref[...]
    t_hi = table.astype(jnp.bfloat16)
    t_lo = (table - t_hi.astype(jnp.float32)).astype(jnp.bfloat16)
    for f in range(_FEW):
        col = idx_ref[:, f]
        onehot = (lax.broadcasted_iota(jnp.int32, (_TC_BLK, _N_TEM), 1)
                  == col[:, None]).astype(jnp.bfloat16)
        out_ref[:, f, :] = (
            jnp.dot(onehot, t_hi, preferred_element_type=jnp.float32)
            + jnp.dot(onehot, t_lo, preferred_element_type=jnp.float32))


_tem_lookup = pl.pallas_call(
    _tem_body,
    grid=(_B // _TC_BLK,),
    in_specs=[
        pl.BlockSpec((_TC_BLK, _FEW), lambda i: (i, 0)),
        pl.BlockSpec((_N_TEM, _D), lambda i: (0, 0)),
    ],
    out_specs=pl.BlockSpec((_TC_BLK, _FEW, _D), lambda i: (i, 0, 0)),
    out_shape=jax.ShapeDtypeStruct((_B, _FEW, _D), jnp.float32),
)


def kernel(idx, idx_tem, ent_emb, tem_emb):
    a = _ent_gather(ent_emb, idx.reshape(-1).astype(jnp.int32))
    b = _tem_lookup(idx_tem.astype(jnp.int32), tem_emb)
    return (a, b)
